# baseline (speedup 1.0000x reference)
import jax
import jax.numpy as jnp
from jax.experimental import pallas as pl
from jax.experimental.pallas import tpu as pltpu

_POOL = 1024
_K = 8
_D = 768
_B = 256
_S = 196
_BLK = 16
_GRID = _B // _BLK


def _stream_body(x_any, x_ref, pe_ref, xsum_ref, sem):
    i = pl.program_id(0)

    @pl.when(i == 0)
    def _():
        def start_one(b, _):
            pltpu.make_async_copy(
                x_any.at[b], pe_ref.at[b, pl.ds(_K, _S), :], sem).start()
            return 0
        jax.lax.fori_loop(0, _B, start_one, 0)

    xsum_ref[...] = jnp.sum(x_ref[...], axis=1)

    @pl.when(i == _GRID - 1)
    def _():
        pltpu.make_async_copy(
            x_any, pe_ref.at[:, pl.ds(_K, _S), :], sem).wait()


def _head_body(xsum_ref, p_ref, sim_ref, idx_ref, rs_ref):
    xm = xsum_ref[...] * (1.0 / _S)
    xn = xm * jax.lax.rsqrt(jnp.maximum(
        jnp.sum(xm * xm, axis=1, keepdims=True), 1e-12))
    p = p_ref[...]
    pn = p * jax.lax.rsqrt(jnp.maximum(
        jnp.sum(p * p, axis=1, keepdims=True), 1e-12))
    sim = jax.lax.dot_general(
        xn, pn, (((1,), (1,)), ((), ())),
        preferred_element_type=jnp.float32)
    sim_ref[...] = sim

    iota = jax.lax.broadcasted_iota(jnp.int32, (_B, _POOL), 1)
    w = sim
    cols = []
    vsum = jnp.float32(0.0)
    for _ in range(_K):
        m = jnp.max(w, axis=1, keepdims=True)
        amax = jnp.min(jnp.where(w == m, iota, _POOL), axis=1,
                       keepdims=True)
        cols.append(amax)
        vsum = vsum + jnp.sum(m)
        w = jnp.where(iota == amax, -jnp.inf, w)
    idx_ref[...] = jnp.concatenate(cols, axis=1)
    rs_ref[0, 0] = vsum * (1.0 / _B)


def _gather_body(idx_ref, p_ref, pe_in_ref, pe_ref, rows_ref, sem):
    def body(r, _):
        b = r // _K
        k = r % _K
        v = idx_ref[b, k]
        rows_ref[b, pl.ds(k, 1), :] = p_ref[pl.ds(v, 1), :]
        return 0

    jax.lax.fori_loop(0, _B * _K, body, 0, unroll=8)
    cp = pltpu.make_async_copy(
        rows_ref, pe_ref.at[:, pl.ds(0, _K), :], sem)
    cp.start()
    cp.wait()


def kernel(x_embed, prompt):
    pe_partial, xsum = pl.pallas_call(
        _stream_body,
        grid=(_GRID,),
        in_specs=[
            pl.BlockSpec(memory_space=pl.ANY),
            pl.BlockSpec((_BLK, _S, _D), lambda i: (i, 0, 0)),
        ],
        out_specs=[
            pl.BlockSpec(memory_space=pl.ANY),
            pl.BlockSpec((_BLK, _D), lambda i: (i, 0)),
        ],
        out_shape=[
            jax.ShapeDtypeStruct((_B, _K + _S, _D), jnp.float32),
            jax.ShapeDtypeStruct((_B, _D), jnp.float32),
        ],
        scratch_shapes=[pltpu.SemaphoreType.DMA],
    )(x_embed, x_embed)

    sim, idx, rs = pl.pallas_call(
        _head_body,
        in_specs=[
            pl.BlockSpec((_B, _D), lambda: (0, 0)),
            pl.BlockSpec((_POOL, _D), lambda: (0, 0)),
        ],
        out_specs=[
            pl.BlockSpec((_B, _POOL), lambda: (0, 0)),
            pl.BlockSpec((_B, _K), lambda: (0, 0)),
            pl.BlockSpec(block_shape=(1, 1), index_map=lambda: (0, 0),
                         memory_space=pltpu.SMEM),
        ],
        out_shape=[
            jax.ShapeDtypeStruct((_B, _POOL), jnp.float32),
            jax.ShapeDtypeStruct((_B, _K), jnp.int32),
            jax.ShapeDtypeStruct((1, 1), jnp.float32),
        ],
    )(xsum, prompt)

    pe = pl.pallas_call(
        _gather_body,
        in_specs=[
            pl.BlockSpec(memory_space=pltpu.SMEM),
            pl.BlockSpec((_POOL, _D), lambda: (0, 0)),
            pl.BlockSpec(memory_space=pl.ANY),
        ],
        out_specs=pl.BlockSpec(memory_space=pl.ANY),
        out_shape=jax.ShapeDtypeStruct((_B, _K + _S, _D), jnp.float32),
        scratch_shapes=[pltpu.VMEM((_B, _K, _D), jnp.float32),
                        pltpu.SemaphoreType.DMA],
        input_output_aliases={2: 0},
    )(idx, prompt, pe_partial)

    return pe, sim, rs.reshape(()), idx


# speedup vs baseline: 12.8434x; 12.8434x over previous
import jax
import jax.numpy as jnp
from jax.experimental import pallas as pl
from jax.experimental.pallas import tpu as pltpu

_POOL = 1024
_K = 8
_D = 768
_B = 256
_S = 196
_BLK = 16
_GRID = _B // _BLK


_NCHUNK = _B // _BLK


def _stream_body(x_any, pe_ref, xsum_ref, bufs, insems, outsems):
    def in_copy(c, buf):
        return pltpu.make_async_copy(
            x_any.at[pl.ds(c * _BLK, _BLK)], bufs.at[buf],
            insems.at[buf])

    def out_copy(c, buf):
        return pltpu.make_async_copy(
            bufs.at[buf],
            pe_ref.at[pl.ds(c * _BLK, _BLK), pl.ds(_K, _S), :],
            outsems.at[buf])

    in_copy(0, 0).start()
    for c in range(_NCHUNK):
        cur = c % 2
        nxt = 1 - cur
        if c + 1 < _NCHUNK:
            if c >= 1:
                out_copy(c - 1, nxt).wait()
            in_copy(c + 1, nxt).start()
        in_copy(c, cur).wait()
        xsum_ref[pl.ds(c * _BLK, _BLK), :] = jnp.sum(bufs[cur], axis=1)
        out_copy(c, cur).start()
    out_copy(_NCHUNK - 2, 0 if (_NCHUNK - 2) % 2 == 0 else 1).wait()
    out_copy(_NCHUNK - 1, 0 if (_NCHUNK - 1) % 2 == 0 else 1).wait()


def _head_body(xsum_ref, p_ref, sim_ref, idx_ref, rs_ref):
    xm = xsum_ref[...] * (1.0 / _S)
    xn = xm * jax.lax.rsqrt(jnp.maximum(
        jnp.sum(xm * xm, axis=1, keepdims=True), 1e-12))
    p = p_ref[...]
    pn = p * jax.lax.rsqrt(jnp.maximum(
        jnp.sum(p * p, axis=1, keepdims=True), 1e-12))
    sim = jax.lax.dot_general(
        xn, pn, (((1,), (1,)), ((), ())),
        preferred_element_type=jnp.float32)
    sim_ref[...] = sim

    iota = jax.lax.broadcasted_iota(jnp.int32, (_B, _POOL), 1)
    w = sim
    cols = []
    vsum = jnp.float32(0.0)
    for _ in range(_K):
        m = jnp.max(w, axis=1, keepdims=True)
        amax = jnp.min(jnp.where(w == m, iota, _POOL), axis=1,
                       keepdims=True)
        cols.append(amax)
        vsum = vsum + jnp.sum(m)
        w = jnp.where(iota == amax, -jnp.inf, w)
    idx_ref[...] = jnp.concatenate(cols, axis=1)
    rs_ref[0, 0] = vsum * (1.0 / _B)


def _gather_body(idx_ref, p_ref, pe_in_ref, pe_ref, rows_ref, sem):
    def body(r, _):
        b = r // _K
        k = r % _K
        v = idx_ref[b, k]
        rows_ref[b, pl.ds(k, 1), :] = p_ref[pl.ds(v, 1), :]
        return 0

    jax.lax.fori_loop(0, _B * _K, body, 0, unroll=8)
    cp = pltpu.make_async_copy(
        rows_ref, pe_ref.at[:, pl.ds(0, _K), :], sem)
    cp.start()
    cp.wait()


def kernel(x_embed, prompt):
    pe_partial, xsum = pl.pallas_call(
        _stream_body,
        in_specs=[pl.BlockSpec(memory_space=pl.ANY)],
        out_specs=[
            pl.BlockSpec(memory_space=pl.ANY),
            pl.BlockSpec((_B, _D), lambda: (0, 0)),
        ],
        out_shape=[
            jax.ShapeDtypeStruct((_B, _K + _S, _D), jnp.float32),
            jax.ShapeDtypeStruct((_B, _D), jnp.float32),
        ],
        scratch_shapes=[
            pltpu.VMEM((2, _BLK, _S, _D), jnp.float32),
            pltpu.SemaphoreType.DMA((2,)),
            pltpu.SemaphoreType.DMA((2,)),
        ],
    )(x_embed)

    sim, idx, rs = pl.pallas_call(
        _head_body,
        in_specs=[
            pl.BlockSpec((_B, _D), lambda: (0, 0)),
            pl.BlockSpec((_POOL, _D), lambda: (0, 0)),
        ],
        out_specs=[
            pl.BlockSpec((_B, _POOL), lambda: (0, 0)),
            pl.BlockSpec((_B, _K), lambda: (0, 0)),
            pl.BlockSpec(block_shape=(1, 1), index_map=lambda: (0, 0),
                         memory_space=pltpu.SMEM),
        ],
        out_shape=[
            jax.ShapeDtypeStruct((_B, _POOL), jnp.float32),
            jax.ShapeDtypeStruct((_B, _K), jnp.int32),
            jax.ShapeDtypeStruct((1, 1), jnp.float32),
        ],
    )(xsum, prompt)

    pe = pl.pallas_call(
        _gather_body,
        in_specs=[
            pl.BlockSpec(memory_space=pltpu.SMEM),
            pl.BlockSpec((_POOL, _D), lambda: (0, 0)),
            pl.BlockSpec(memory_space=pl.ANY),
        ],
        out_specs=pl.BlockSpec(memory_space=pl.ANY),
        out_shape=jax.ShapeDtypeStruct((_B, _K + _S, _D), jnp.float32),
        scratch_shapes=[pltpu.VMEM((_B, _K, _D), jnp.float32),
                        pltpu.SemaphoreType.DMA],
        input_output_aliases={2: 0},
    )(idx, prompt, pe_partial)

    return pe, sim, rs.reshape(()), idx


# speedup vs baseline: 12.8474x; 1.0003x over previous
import jax
import jax.numpy as jnp
from jax.experimental import pallas as pl
from jax.experimental.pallas import tpu as pltpu

_POOL = 1024
_K = 8
_D = 768
_B = 256
_S = 196
_BLK = 16
_GRID = _B // _BLK


_CB = 8
_NCHUNK = _B // _CB
_NBUF = 4


def _stream_body(x_any, pe_ref, xsum_ref, bufs, insems, outsems):
    def in_copy(c, buf):
        return pltpu.make_async_copy(
            x_any.at[pl.ds(c * _CB, _CB)], bufs.at[buf],
            insems.at[buf])

    def out_copy(c, buf):
        return pltpu.make_async_copy(
            bufs.at[buf],
            pe_ref.at[pl.ds(c * _CB, _CB), pl.ds(_K, _S), :],
            outsems.at[buf])

    for b in range(_NBUF - 1):
        in_copy(b, b).start()
    for i in range(_NCHUNK):
        if i + _NBUF - 1 < _NCHUNK:
            if i >= 1:
                out_copy(i - 1, (i - 1) % _NBUF).wait()
            in_copy(i + _NBUF - 1, (i + _NBUF - 1) % _NBUF).start()
        in_copy(i, i % _NBUF).wait()
        xsum_ref[pl.ds(i * _CB, _CB), :] = jnp.sum(bufs[i % _NBUF], axis=1)
        out_copy(i, i % _NBUF).start()
    for c in range(_NCHUNK - _NBUF, _NCHUNK):
        out_copy(c, c % _NBUF).wait()


def _head_body(xsum_ref, p_ref, sim_ref, idx_ref, rs_ref):
    xm = xsum_ref[...] * (1.0 / _S)
    xn = xm * jax.lax.rsqrt(jnp.maximum(
        jnp.sum(xm * xm, axis=1, keepdims=True), 1e-12))
    p = p_ref[...]
    pn = p * jax.lax.rsqrt(jnp.maximum(
        jnp.sum(p * p, axis=1, keepdims=True), 1e-12))
    sim = jax.lax.dot_general(
        xn, pn, (((1,), (1,)), ((), ())),
        preferred_element_type=jnp.float32)
    sim_ref[...] = sim

    iota = jax.lax.broadcasted_iota(jnp.int32, (_B, _POOL), 1)
    w = sim
    cols = []
    vsum = jnp.float32(0.0)
    for _ in range(_K):
        m = jnp.max(w, axis=1, keepdims=True)
        amax = jnp.min(jnp.where(w == m, iota, _POOL), axis=1,
                       keepdims=True)
        cols.append(amax)
        vsum = vsum + jnp.sum(m)
        w = jnp.where(iota == amax, -jnp.inf, w)
    idx_ref[...] = jnp.concatenate(cols, axis=1)
    rs_ref[0, 0] = vsum * (1.0 / _B)


def _gather_body(idx_ref, p_ref, pe_in_ref, pe_ref, rows_ref, sem):
    def body(r, _):
        b = r // _K
        k = r % _K
        v = idx_ref[b, k]
        rows_ref[b, pl.ds(k, 1), :] = p_ref[pl.ds(v, 1), :]
        return 0

    jax.lax.fori_loop(0, _B * _K, body, 0, unroll=8)
    cp = pltpu.make_async_copy(
        rows_ref, pe_ref.at[:, pl.ds(0, _K), :], sem)
    cp.start()
    cp.wait()


def kernel(x_embed, prompt):
    pe_partial, xsum = pl.pallas_call(
        _stream_body,
        in_specs=[pl.BlockSpec(memory_space=pl.ANY)],
        out_specs=[
            pl.BlockSpec(memory_space=pl.ANY),
            pl.BlockSpec((_B, _D), lambda: (0, 0)),
        ],
        out_shape=[
            jax.ShapeDtypeStruct((_B, _K + _S, _D), jnp.float32),
            jax.ShapeDtypeStruct((_B, _D), jnp.float32),
        ],
        scratch_shapes=[
            pltpu.VMEM((_NBUF, _CB, _S, _D), jnp.float32),
            pltpu.SemaphoreType.DMA((_NBUF,)),
            pltpu.SemaphoreType.DMA((_NBUF,)),
        ],
    )(x_embed)

    sim, idx, rs = pl.pallas_call(
        _head_body,
        in_specs=[
            pl.BlockSpec((_B, _D), lambda: (0, 0)),
            pl.BlockSpec((_POOL, _D), lambda: (0, 0)),
        ],
        out_specs=[
            pl.BlockSpec((_B, _POOL), lambda: (0, 0)),
            pl.BlockSpec((_B, _K), lambda: (0, 0)),
            pl.BlockSpec(block_shape=(1, 1), index_map=lambda: (0, 0),
                         memory_space=pltpu.SMEM),
        ],
        out_shape=[
            jax.ShapeDtypeStruct((_B, _POOL), jnp.float32),
            jax.ShapeDtypeStruct((_B, _K), jnp.int32),
            jax.ShapeDtypeStruct((1, 1), jnp.float32),
        ],
    )(xsum, prompt)

    pe = pl.pallas_call(
        _gather_body,
        in_specs=[
            pl.BlockSpec(memory_space=pltpu.SMEM),
            pl.BlockSpec((_POOL, _D), lambda: (0, 0)),
            pl.BlockSpec(memory_space=pl.ANY),
        ],
        out_specs=pl.BlockSpec(memory_space=pl.ANY),
        out_shape=jax.ShapeDtypeStruct((_B, _K + _S, _D), jnp.float32),
        scratch_shapes=[pltpu.VMEM((_B, _K, _D), jnp.float32),
                        pltpu.SemaphoreType.DMA],
        input_output_aliases={2: 0},
    )(idx, prompt, pe_partial)

    return pe, sim, rs.reshape(()), idx
